# baseline (speedup 1.0000x reference)
import functools

import jax
import jax.numpy as jnp
from jax import lax
from jax.experimental import pallas as pl
from jax.experimental.pallas import tpu as pltpu
from jax.experimental.pallas import tpu_sc as plsc

_N = 10000
_E = 320000
_DP = 128
_NP = 10240
_W = 128
_NWIN = _E // _W
_WPW = 80
_IPAD = _WPW * 32
_BE = 2000


def _relu(x):
    return jnp.maximum(x, 0.0)


def _dot(a, b):
    return jnp.dot(a, b, preferred_element_type=jnp.float32)


def _pw(w, rows, cols):
    return jnp.pad(w, ((0, rows - w.shape[0]), (0, cols - w.shape[1])))


def _pb(b, n):
    return jnp.pad(b, (0, n - b.shape[0])).reshape(1, n)


def _sc_mesh():
    return plsc.VectorSubcoreMesh(core_axis_name="core", subcore_axis_name="subcore")


def _gather_pair(p1, p2, r2d, s2d):
    rows_t = pltpu.VMEM((_W, _DP), jnp.float32)

    @functools.partial(
        pl.kernel,
        out_type=jax.ShapeDtypeStruct((_E, _DP), jnp.float32),
        mesh=_sc_mesh(),
        scratch_types=[
            pltpu.VMEM((_WPW, _W), jnp.int32),
            pltpu.VMEM((_WPW, _W), jnp.int32),
            rows_t, rows_t, rows_t, rows_t,
            rows_t, rows_t,
            pltpu.SemaphoreType.DMA, pltpu.SemaphoreType.DMA,
            pltpu.SemaphoreType.DMA, pltpu.SemaphoreType.DMA,
            pltpu.SemaphoreType.DMA, pltpu.SemaphoreType.DMA,
            pltpu.SemaphoreType.DMA,
        ],
    )
    def k(p1_hbm, p2_hbm, r_hbm, s_hbm, g_hbm,
          ir, isb, r1a, r2a, r1b, r2b, oa, ob,
          s1a, s2a, s1b, s2b, soa, sob, sp):
        cid = lax.axis_index("core")
        sid = lax.axis_index("subcore")
        wid = sid * 2 + cid
        base = wid * _WPW

        pltpu.async_copy(r_hbm.at[pl.ds(base, _WPW)], ir, sp).wait()
        pltpu.async_copy(s_hbm.at[pl.ds(base, _WPW)], isb, sp).wait()

        bufs = ((r1a, r2a, s1a, s2a, oa, soa),
                (r1b, r2b, s1b, s2b, ob, sob))

        def issue(t, r1, r2, s1, s2):
            @pl.when(jnp.logical_and(t < _WPW, base + t < _NWIN))
            def _():
                pltpu.async_copy(p1_hbm.at[ir.at[t]], r1, s1)
                pltpu.async_copy(p2_hbm.at[isb.at[t]], r2, s2)

        def wait_rows(t, r1, r2, s1, s2):
            @pl.when(base + t < _NWIN)
            def _():
                pltpu.make_async_copy(p1_hbm.at[ir.at[t]], r1, s1).wait()
                pltpu.make_async_copy(p2_hbm.at[isb.at[t]], r2, s2).wait()

        def wait_out(t, o, so):
            @pl.when(jnp.logical_and(t >= 0, base + t < _NWIN))
            def _():
                pltpu.make_async_copy(
                    o, g_hbm.at[pl.ds((base + t) * _W, _W)], so).wait()

        issue(0, r1a, r2a, s1a, s2a)

        @pl.loop(0, _WPW // 2)
        def _(u):
            for half in range(2):
                t = u * 2 + half
                r1, r2, s1, s2, o, so = bufs[half]
                n1, n2, ns1, ns2, _, _ = bufs[1 - half]
                issue(t + 1, n1, n2, ns1, ns2)
                wait_rows(t, r1, r2, s1, s2)
                wait_out(t - 2, o, so)

                @pl.when(base + t < _NWIN)
                def _():
                    @pl.loop(0, _W)
                    def _(i):
                        for c in range(_DP // 16):
                            slc = (pl.ds(i, 1), pl.ds(c * 16, 16))
                            o.at[slc][...] = r1.at[slc][...] + r2.at[slc][...]

                    pltpu.async_copy(
                        o, g_hbm.at[pl.ds((base + t) * _W, _W)], so)

        wait_out(_WPW - 2, oa, soa)
        wait_out(_WPW - 1, ob, sob)

    return k(p1, p2, r2d, s2d)


def _scatter_add(contrib, r1d):
    @functools.partial(
        pl.kernel,
        out_type=jax.ShapeDtypeStruct((2, _NP, _DP), jnp.float32),
        mesh=_sc_mesh(),
        scratch_types=[
            pltpu.VMEM((32, _DP), jnp.float32),
            pltpu.VMEM((_W, _DP), jnp.float32),
            pltpu.VMEM((_W, _DP), jnp.float32),
            pltpu.VMEM((_WPW, _W), jnp.int32),
            pltpu.VMEM_SHARED((_NP, _DP), jnp.float32),
            pltpu.SemaphoreType.DMA, pltpu.SemaphoreType.DMA,
            pltpu.SemaphoreType.DMA,
        ],
    )
    def k(c_hbm, r_hbm, out_hbm, zbuf, cba, cbb, ir, agg, sa, sb, sp):
        cid = lax.axis_index("core")
        sid = lax.axis_index("subcore")
        wid = sid * 2 + cid
        base = wid * _WPW

        pltpu.async_copy(r_hbm.at[pl.ds(base, _WPW)], ir, sp).wait()

        @pl.loop(0, 32)
        def _(i):
            for c in range(_DP // 16):
                zbuf.at[pl.ds(i, 1), pl.ds(c * 16, 16)][...] = jnp.zeros(
                    (1, 16), jnp.float32)

        @pl.loop(0, 20)
        def _(j):
            pltpu.sync_copy(zbuf, agg.at[pl.ds(sid * 640 + j * 32, 32)])

        plsc.subcore_barrier()

        def issue(t, cb, sem):
            @pl.when(jnp.logical_and(t < _WPW, base + t < _NWIN))
            def _():
                pltpu.async_copy(
                    c_hbm.at[pl.ds((base + t) * _W, _W)], cb, sem)

        def wait_load(t, cb, sem):
            @pl.when(base + t < _NWIN)
            def _():
                pltpu.make_async_copy(
                    c_hbm.at[pl.ds((base + t) * _W, _W)], cb, sem).wait()

        issue(0, cba, sa)

        @pl.loop(0, _WPW // 2)
        def _(u):
            for half in range(2):
                t = u * 2 + half
                cb, sem = (cba, sa) if half == 0 else (cbb, sb)
                ncb, nsem = (cbb, sb) if half == 0 else (cba, sa)
                issue(t + 1, ncb, nsem)
                wait_load(t, cb, sem)

                @pl.when(base + t < _NWIN)
                def _():
                    pltpu.sync_copy(cb, agg.at[ir.at[t]], add=True)

        plsc.subcore_barrier()

        @pl.loop(0, 10)
        def _(j):
            rowo = sid * 640 + j * 64
            pltpu.sync_copy(agg.at[pl.ds(rowo, 64)], out_hbm.at[cid, pl.ds(rowo, 64)])

    return k(contrib, r1d)


def _full(shape):
    nd = len(shape)
    return pl.BlockSpec(shape, lambda i: (0,) * nd)


def _node_encode(nodes, w1, b1, w2, b2, wp1, wp2):
    def body(x, w1r, b1r, w2r, b2r, wp1r, wp2r, n0o, p1o, p2o):
        h = _relu(_dot(x[...], w1r[...]) + b1r[...])
        n0 = _relu(_dot(h, w2r[...]) + b2r[...])
        n0o[...] = n0
        p1o[...] = _dot(n0, wp1r[...])
        p2o[...] = _dot(n0, wp2r[...])

    out = jax.ShapeDtypeStruct((_N, _DP), jnp.float32)
    return pl.pallas_call(
        body,
        out_shape=[out, out, out],
    )(nodes, w1, b1, w2, b2, wp1, wp2)


def _node_update(n_prev, parts, wn1, bn, wn2, wp1, wp2):
    def body(n, pr, wn1r, bnr, wn2r, wp1r, wp2r, n1o, p1o, p2o):
        agg = pr[0, : _N, :] + pr[1, : _N, :]
        n1 = _relu(_dot(n[...], wn1r[...]) + _dot(agg, wn2r[...]) + bnr[...])
        n1o[...] = n1
        p1o[...] = _dot(n1, wp1r[...])
        p2o[...] = _dot(n1, wp2r[...])

    out = jax.ShapeDtypeStruct((_N, _DP), jnp.float32)
    return pl.pallas_call(
        body,
        out_shape=[out, out, out],
    )(n_prev, parts, wn1, bn, wn2, wp1, wp2)


def _node_final(n_prev, parts, wn1, bn, wn2, d1, bd1, d2, bd2, won, bon):
    def body(n, pr, wn1r, bnr, wn2r, d1r, bd1r, d2r, bd2r, wonr, bonr, outo):
        agg = pr[0, : _N, :] + pr[1, : _N, :]
        n2 = _relu(_dot(n[...], wn1r[...]) + _dot(agg, wn2r[...]) + bnr[...])
        dn = _relu(_dot(n2, d1r[...]) + bd1r[...])
        dn = _relu(_dot(dn, d2r[...]) + bd2r[...])
        outo[...] = _dot(dn, wonr[...]) + bonr[...]

    return pl.pallas_call(
        body,
        out_shape=jax.ShapeDtypeStruct((_N, 3), jnp.float32),
    )(n_prev, parts, wn1, bn, wn2, d1, bd1, d2, bd2, won, bon)


def _enc_edge(efeat, we1, be1, we2, be2, w30, b30):
    bf = jnp.bfloat16

    def body(f, we1r, be1r, we2r, be2r, w30r, b30r, t0o):
        h = _relu(_dot(f[...].astype(bf), we1r[...].astype(bf)) + be1r[...])
        ef0 = _relu(_dot(h.astype(bf), we2r[...].astype(bf)) + be2r[...])
        t0o[...] = (_dot(ef0.astype(bf), w30r[...].astype(bf))
                    + b30r[...]).astype(bf)

    return pl.pallas_call(
        body,
        grid=(_E // _BE,),
        in_specs=[
            pl.BlockSpec((_BE, 16), lambda i: (i, 0)),
            _full((16, 32)), _full((1, 32)),
            _full((32, _DP)), _full((1, _DP)),
            _full((_DP, _DP)), _full((1, _DP)),
        ],
        out_specs=pl.BlockSpec((_BE, _DP), lambda i: (i, 0)),
        out_shape=jax.ShapeDtypeStruct((_E, _DP), jnp.bfloat16),
    )(efeat, we1, be1, we2, be2, w30, b30)


def _edge_update(G, T, ew):
    def body(g, t, w, co, efo):
        ef = _relu(g[...] + t[...].astype(jnp.float32))
        co[...] = ef * w[...]
        efo[...] = ef.astype(jnp.bfloat16)

    return pl.pallas_call(
        body,
        grid=(_E // _BE,),
        in_specs=[
            pl.BlockSpec((_BE, _DP), lambda i: (i, 0)),
            pl.BlockSpec((_BE, _DP), lambda i: (i, 0)),
            pl.BlockSpec((_BE, 1), lambda i: (i, 0)),
        ],
        out_specs=[
            pl.BlockSpec((_BE, _DP), lambda i: (i, 0)),
            pl.BlockSpec((_BE, _DP), lambda i: (i, 0)),
        ],
        out_shape=[
            jax.ShapeDtypeStruct((_E, _DP), jnp.float32),
            jax.ShapeDtypeStruct((_E, _DP), jnp.bfloat16),
        ],
    )(G, T, ew)


def _edge_proj(efb, w31, b31):
    bf = jnp.bfloat16

    def body(f, w31r, b31r, t1o):
        t1o[...] = (_dot(f[...], w31r[...].astype(bf)) + b31r[...]).astype(bf)

    return pl.pallas_call(
        body,
        grid=(_E // _BE,),
        in_specs=[
            pl.BlockSpec((_BE, _DP), lambda i: (i, 0)),
            _full((_DP, _DP)), _full((1, _DP)),
        ],
        out_specs=pl.BlockSpec((_BE, _DP), lambda i: (i, 0)),
        out_shape=jax.ShapeDtypeStruct((_E, _DP), jnp.bfloat16),
    )(efb, w31, b31)


def _dec_edge(efb, d1, bd1, d2, bd2, woe, boe):
    bf = jnp.bfloat16

    def body(f, d1r, bd1r, d2r, bd2r, woer, boer, oeo):
        de = _relu(_dot(f[...], d1r[...].astype(bf)) + bd1r[...])
        de = _relu(_dot(de.astype(bf), d2r[...].astype(bf)) + bd2r[...])
        oeo[...] = _dot(de.astype(bf), woer[...].astype(bf)) + boer[...]

    return pl.pallas_call(
        body,
        grid=(_E // _BE,),
        in_specs=[
            pl.BlockSpec((_BE, _DP), lambda i: (i, 0)),
            _full((_DP, _DP)), _full((1, _DP)),
            _full((_DP, 32)), _full((1, 32)),
            _full((32, 3)), _full((1, 3)),
        ],
        out_specs=pl.BlockSpec((_BE, 3), lambda i: (i, 0)),
        out_shape=jax.ShapeDtypeStruct((_E, 3), jnp.float32),
    )(efb, d1, bd1, d2, bd2, woe, boe)


def kernel(nodes, edge_features, edges, edge_weights,
           enc_n1_w, enc_n1_b, enc_n2_w, enc_n2_b,
           enc_e1_w, enc_e1_b, enc_e2_w, enc_e2_b,
           g0_edge_w, g0_edge_b, g0_node_w, g0_node_b,
           g1_edge_w, g1_edge_b, g1_node_w, g1_node_b,
           dec_n1_w, dec_n1_b, dec_n2_w, dec_n2_b,
           out_n_w, out_n_b, out_e_w, out_e_b):
    r = edges[:, 1]
    s = edges[:, 0]
    pad = _IPAD * _W - _E
    rw = jnp.pad(r, (0, pad)).reshape(_IPAD, _W)
    sw = jnp.pad(s, (0, pad)).reshape(_IPAD, _W)

    g0w1 = _pw(g0_edge_w[0:72], _DP, _DP)
    g0w2 = _pw(g0_edge_w[72:144], _DP, _DP)
    g0w3 = _pw(g0_edge_w[144:216], _DP, _DP)
    g0b3 = _pb(g0_edge_b, _DP)
    g1w1 = _pw(g1_edge_w[0:72], _DP, _DP)
    g1w2 = _pw(g1_edge_w[72:144], _DP, _DP)
    g1w3 = _pw(g1_edge_w[144:216], _DP, _DP)
    g1b3 = _pb(g1_edge_b, _DP)
    g0n1 = _pw(g0_node_w[0:72], _DP, _DP)
    g0n2 = _pw(g0_node_w[72:144], _DP, _DP)
    g0nb = _pb(g0_node_b, _DP)
    g1n1 = _pw(g1_node_w[0:72], _DP, _DP)
    g1n2 = _pw(g1_node_w[72:144], _DP, _DP)
    g1nb = _pb(g1_node_b, _DP)
    d1 = _pw(dec_n1_w, _DP, _DP)
    bd1 = _pb(dec_n1_b, _DP)
    d2 = _pw(dec_n2_w, _DP, 32)
    bd2 = _pb(dec_n2_b, 32)

    n0, p1_0, p2_0 = _node_encode(
        nodes, enc_n1_w, enc_n1_b.reshape(1, 32),
        _pw(enc_n2_w, 32, _DP), _pb(enc_n2_b, _DP), g0w1, g0w2)

    T0 = _enc_edge(
        edge_features, enc_e1_w, enc_e1_b.reshape(1, 32),
        _pw(enc_e2_w, 32, _DP), _pb(enc_e2_b, _DP), g0w3, g0b3)
    G0 = _gather_pair(p1_0, p2_0, rw, sw)
    contrib1, ef1b = _edge_update(G0, T0, edge_weights)
    parts0 = _scatter_add(contrib1, rw)
    T1 = _edge_proj(ef1b, g1w3, g1b3)
    n1, p1_1, p2_1 = _node_update(n0, parts0, g0n1, g0nb, g0n2, g1w1, g1w2)

    G1 = _gather_pair(p1_1, p2_1, rw, sw)
    contrib2, ef2b = _edge_update(G1, T1, edge_weights)
    parts1 = _scatter_add(contrib2, rw)
    out_e = _dec_edge(ef2b, d1, bd1, d2, bd2, out_e_w, _pb(out_e_b, 3))

    out_n = _node_final(
        n1, parts1, g1n1, g1nb, g1n2, d1, bd1, d2, bd2,
        out_n_w, _pb(out_n_b, 3))

    return out_n, out_e


# speedup vs baseline: 1.3313x; 1.3313x over previous
import functools

import jax
import jax.numpy as jnp
from jax import lax
from jax.experimental import pallas as pl
from jax.experimental.pallas import tpu as pltpu
from jax.experimental.pallas import tpu_sc as plsc

_N = 10000
_E = 320000
_DP = 128
_NP = 10240
_W = 128
_NWIN = _E // _W
_WPW = 80
_IPAD = _WPW * 32
_BE = 4000


def _relu(x):
    return jnp.maximum(x, 0.0)


def _dot(a, b):
    return jnp.dot(a, b, preferred_element_type=jnp.float32)


def _pw(w, rows, cols):
    return jnp.pad(w, ((0, rows - w.shape[0]), (0, cols - w.shape[1])))


def _pb(b, n):
    return jnp.pad(b, (0, n - b.shape[0])).reshape(1, n)


def _sc_mesh():
    return plsc.VectorSubcoreMesh(core_axis_name="core", subcore_axis_name="subcore")


def _gather_pair(p1, p2, r2d, s2d):
    rows_t = pltpu.VMEM((_W, _DP), jnp.float32)

    @functools.partial(
        pl.kernel,
        out_type=jax.ShapeDtypeStruct((_E, _DP), jnp.float32),
        mesh=_sc_mesh(),
        scratch_types=[
            pltpu.VMEM((_WPW, _W), jnp.int32),
            pltpu.VMEM((_WPW, _W), jnp.int32),
            rows_t, rows_t, rows_t, rows_t,
            rows_t, rows_t,
            pltpu.SemaphoreType.DMA, pltpu.SemaphoreType.DMA,
            pltpu.SemaphoreType.DMA, pltpu.SemaphoreType.DMA,
            pltpu.SemaphoreType.DMA, pltpu.SemaphoreType.DMA,
            pltpu.SemaphoreType.DMA,
        ],
    )
    def k(p1_hbm, p2_hbm, r_hbm, s_hbm, g_hbm,
          ir, isb, r1a, r2a, r1b, r2b, oa, ob,
          s1a, s2a, s1b, s2b, soa, sob, sp):
        cid = lax.axis_index("core")
        sid = lax.axis_index("subcore")
        wid = sid * 2 + cid
        base = wid * _WPW

        pltpu.async_copy(r_hbm.at[pl.ds(base, _WPW)], ir, sp).wait()
        pltpu.async_copy(s_hbm.at[pl.ds(base, _WPW)], isb, sp).wait()

        bufs = ((r1a, r2a, s1a, s2a, oa, soa),
                (r1b, r2b, s1b, s2b, ob, sob))

        def issue(t, r1, r2, s1, s2):
            @pl.when(jnp.logical_and(t < _WPW, base + t < _NWIN))
            def _():
                pltpu.async_copy(p1_hbm.at[ir.at[t]], r1, s1)
                pltpu.async_copy(p2_hbm.at[isb.at[t]], r2, s2)

        def wait_rows(t, r1, r2, s1, s2):
            @pl.when(base + t < _NWIN)
            def _():
                pltpu.make_async_copy(p1_hbm.at[ir.at[t]], r1, s1).wait()
                pltpu.make_async_copy(p2_hbm.at[isb.at[t]], r2, s2).wait()

        def wait_out(t, o, so):
            @pl.when(jnp.logical_and(t >= 0, base + t < _NWIN))
            def _():
                pltpu.make_async_copy(
                    o, g_hbm.at[pl.ds((base + t) * _W, _W)], so).wait()

        issue(0, r1a, r2a, s1a, s2a)

        @pl.loop(0, _WPW // 2)
        def _(u):
            for half in range(2):
                t = u * 2 + half
                r1, r2, s1, s2, o, so = bufs[half]
                n1, n2, ns1, ns2, _, _ = bufs[1 - half]
                issue(t + 1, n1, n2, ns1, ns2)
                wait_rows(t, r1, r2, s1, s2)
                wait_out(t - 2, o, so)

                @pl.when(base + t < _NWIN)
                def _():
                    @pl.loop(0, _W)
                    def _(i):
                        for c in range(_DP // 16):
                            slc = (pl.ds(i, 1), pl.ds(c * 16, 16))
                            o.at[slc][...] = r1.at[slc][...] + r2.at[slc][...]

                    pltpu.async_copy(
                        o, g_hbm.at[pl.ds((base + t) * _W, _W)], so)

        wait_out(_WPW - 2, oa, soa)
        wait_out(_WPW - 1, ob, sob)

    return k(p1, p2, r2d, s2d)


def _scatter_add(contrib, r1d):
    @functools.partial(
        pl.kernel,
        out_type=jax.ShapeDtypeStruct((2, _NP, _DP), jnp.float32),
        mesh=_sc_mesh(),
        scratch_types=[
            pltpu.VMEM((32, _DP), jnp.float32),
            pltpu.VMEM((_W, _DP), jnp.float32),
            pltpu.VMEM((_W, _DP), jnp.float32),
            pltpu.VMEM((_WPW, _W), jnp.int32),
            pltpu.VMEM_SHARED((_NP, _DP), jnp.float32),
            pltpu.SemaphoreType.DMA, pltpu.SemaphoreType.DMA,
            pltpu.SemaphoreType.DMA,
        ],
    )
    def k(c_hbm, r_hbm, out_hbm, zbuf, cba, cbb, ir, agg, sa, sb, sp):
        cid = lax.axis_index("core")
        sid = lax.axis_index("subcore")
        wid = sid * 2 + cid
        base = wid * _WPW

        pltpu.async_copy(r_hbm.at[pl.ds(base, _WPW)], ir, sp).wait()

        @pl.loop(0, 32)
        def _(i):
            for c in range(_DP // 16):
                zbuf.at[pl.ds(i, 1), pl.ds(c * 16, 16)][...] = jnp.zeros(
                    (1, 16), jnp.float32)

        @pl.loop(0, 20)
        def _(j):
            pltpu.sync_copy(zbuf, agg.at[pl.ds(sid * 640 + j * 32, 32)])

        plsc.subcore_barrier()

        def issue(t, cb, sem):
            @pl.when(jnp.logical_and(t < _WPW, base + t < _NWIN))
            def _():
                pltpu.async_copy(
                    c_hbm.at[pl.ds((base + t) * _W, _W)], cb, sem)

        def wait_load(t, cb, sem):
            @pl.when(base + t < _NWIN)
            def _():
                pltpu.make_async_copy(
                    c_hbm.at[pl.ds((base + t) * _W, _W)], cb, sem).wait()

        issue(0, cba, sa)

        @pl.loop(0, _WPW // 2)
        def _(u):
            for half in range(2):
                t = u * 2 + half
                cb, sem = (cba, sa) if half == 0 else (cbb, sb)
                ncb, nsem = (cbb, sb) if half == 0 else (cba, sa)
                issue(t + 1, ncb, nsem)
                wait_load(t, cb, sem)

                @pl.when(base + t < _NWIN)
                def _():
                    pltpu.sync_copy(cb, agg.at[ir.at[t]], add=True)

        plsc.subcore_barrier()

        @pl.loop(0, 10)
        def _(j):
            rowo = sid * 640 + j * 64
            pltpu.sync_copy(agg.at[pl.ds(rowo, 64)], out_hbm.at[cid, pl.ds(rowo, 64)])

    return k(contrib, r1d)


def _full(shape):
    nd = len(shape)
    return pl.BlockSpec(shape, lambda i: (0,) * nd)


def _node_encode(nodes, w1, b1, w2, b2, wp1, wp2):
    def body(x, w1r, b1r, w2r, b2r, wp1r, wp2r, n0o, p1o, p2o):
        h = _relu(_dot(x[...], w1r[...]) + b1r[...])
        n0 = _relu(_dot(h, w2r[...]) + b2r[...])
        n0o[...] = n0
        p1o[...] = _dot(n0, wp1r[...])
        p2o[...] = _dot(n0, wp2r[...])

    out = jax.ShapeDtypeStruct((_N, _DP), jnp.float32)
    return pl.pallas_call(
        body,
        out_shape=[out, out, out],
    )(nodes, w1, b1, w2, b2, wp1, wp2)


def _node_update(n_prev, parts, wn1, bn, wn2, wp1, wp2):
    def body(n, pr, wn1r, bnr, wn2r, wp1r, wp2r, n1o, p1o, p2o):
        agg = pr[0, : _N, :] + pr[1, : _N, :]
        n1 = _relu(_dot(n[...], wn1r[...]) + _dot(agg, wn2r[...]) + bnr[...])
        n1o[...] = n1
        p1o[...] = _dot(n1, wp1r[...])
        p2o[...] = _dot(n1, wp2r[...])

    out = jax.ShapeDtypeStruct((_N, _DP), jnp.float32)
    return pl.pallas_call(
        body,
        out_shape=[out, out, out],
    )(n_prev, parts, wn1, bn, wn2, wp1, wp2)


def _node_final(n_prev, parts, wn1, bn, wn2, d1, bd1, d2, bd2, won, bon):
    def body(n, pr, wn1r, bnr, wn2r, d1r, bd1r, d2r, bd2r, wonr, bonr, outo):
        agg = pr[0, : _N, :] + pr[1, : _N, :]
        n2 = _relu(_dot(n[...], wn1r[...]) + _dot(agg, wn2r[...]) + bnr[...])
        dn = _relu(_dot(n2, d1r[...]) + bd1r[...])
        dn = _relu(_dot(dn, d2r[...]) + bd2r[...])
        outo[...] = _dot(dn, wonr[...]) + bonr[...]

    return pl.pallas_call(
        body,
        out_shape=jax.ShapeDtypeStruct((_N, 3), jnp.float32),
    )(n_prev, parts, wn1, bn, wn2, d1, bd1, d2, bd2, won, bon)


def _edge_layer0(G0, efeat, ew, we1, be1, we2, be2, w30, b30, w31, b31):
    bf = jnp.bfloat16

    def body(g, f, w, we1r, be1r, we2r, be2r, w30r, b30r, w31r, b31r, co, t1o):
        h = _relu(_dot(f[...].astype(bf), we1r[...].astype(bf)) + be1r[...])
        ef0 = _relu(_dot(h.astype(bf), we2r[...].astype(bf)) + be2r[...])
        ef1 = _relu(g[...] + _dot(ef0.astype(bf), w30r[...].astype(bf))
                    + b30r[...])
        co[...] = ef1 * w[...]
        t1o[...] = (_dot(ef1.astype(bf), w31r[...].astype(bf))
                    + b31r[...]).astype(bf)

    return pl.pallas_call(
        body,
        grid=(_E // _BE,),
        in_specs=[
            pl.BlockSpec((_BE, _DP), lambda i: (i, 0)),
            pl.BlockSpec((_BE, 16), lambda i: (i, 0)),
            pl.BlockSpec((_BE, 1), lambda i: (i, 0)),
            _full((16, 32)), _full((1, 32)),
            _full((32, _DP)), _full((1, _DP)),
            _full((_DP, _DP)), _full((1, _DP)),
            _full((_DP, _DP)), _full((1, _DP)),
        ],
        out_specs=[
            pl.BlockSpec((_BE, _DP), lambda i: (i, 0)),
            pl.BlockSpec((_BE, _DP), lambda i: (i, 0)),
        ],
        out_shape=[
            jax.ShapeDtypeStruct((_E, _DP), jnp.float32),
            jax.ShapeDtypeStruct((_E, _DP), jnp.bfloat16),
        ],
    )(G0, efeat, ew, we1, be1, we2, be2, w30, b30, w31, b31)


def _edge_layer1(G1, T1, ew, d1, bd1, d2, bd2, woe, boe):
    bf = jnp.bfloat16

    def body(g, t, w, d1r, bd1r, d2r, bd2r, woer, boer, co, oeo):
        ef2 = _relu(g[...] + t[...].astype(jnp.float32))
        co[...] = ef2 * w[...]
        de = _relu(_dot(ef2.astype(bf), d1r[...].astype(bf)) + bd1r[...])
        de = _relu(_dot(de.astype(bf), d2r[...].astype(bf)) + bd2r[...])
        oeo[...] = _dot(de.astype(bf), woer[...].astype(bf)) + boer[...]

    return pl.pallas_call(
        body,
        grid=(_E // _BE,),
        in_specs=[
            pl.BlockSpec((_BE, _DP), lambda i: (i, 0)),
            pl.BlockSpec((_BE, _DP), lambda i: (i, 0)),
            pl.BlockSpec((_BE, 1), lambda i: (i, 0)),
            _full((_DP, _DP)), _full((1, _DP)),
            _full((_DP, 32)), _full((1, 32)),
            _full((32, 3)), _full((1, 3)),
        ],
        out_specs=[
            pl.BlockSpec((_BE, _DP), lambda i: (i, 0)),
            pl.BlockSpec((_BE, 3), lambda i: (i, 0)),
        ],
        out_shape=[
            jax.ShapeDtypeStruct((_E, _DP), jnp.float32),
            jax.ShapeDtypeStruct((_E, 3), jnp.float32),
        ],
    )(G1, T1, ew, d1, bd1, d2, bd2, woe, boe)


def kernel(nodes, edge_features, edges, edge_weights,
           enc_n1_w, enc_n1_b, enc_n2_w, enc_n2_b,
           enc_e1_w, enc_e1_b, enc_e2_w, enc_e2_b,
           g0_edge_w, g0_edge_b, g0_node_w, g0_node_b,
           g1_edge_w, g1_edge_b, g1_node_w, g1_node_b,
           dec_n1_w, dec_n1_b, dec_n2_w, dec_n2_b,
           out_n_w, out_n_b, out_e_w, out_e_b):
    r = edges[:, 1]
    s = edges[:, 0]
    pad = _IPAD * _W - _E
    rw = jnp.pad(r, (0, pad)).reshape(_IPAD, _W)
    sw = jnp.pad(s, (0, pad)).reshape(_IPAD, _W)

    g0w1 = _pw(g0_edge_w[0:72], _DP, _DP)
    g0w2 = _pw(g0_edge_w[72:144], _DP, _DP)
    g0w3 = _pw(g0_edge_w[144:216], _DP, _DP)
    g0b3 = _pb(g0_edge_b, _DP)
    g1w1 = _pw(g1_edge_w[0:72], _DP, _DP)
    g1w2 = _pw(g1_edge_w[72:144], _DP, _DP)
    g1w3 = _pw(g1_edge_w[144:216], _DP, _DP)
    g1b3 = _pb(g1_edge_b, _DP)
    g0n1 = _pw(g0_node_w[0:72], _DP, _DP)
    g0n2 = _pw(g0_node_w[72:144], _DP, _DP)
    g0nb = _pb(g0_node_b, _DP)
    g1n1 = _pw(g1_node_w[0:72], _DP, _DP)
    g1n2 = _pw(g1_node_w[72:144], _DP, _DP)
    g1nb = _pb(g1_node_b, _DP)
    d1 = _pw(dec_n1_w, _DP, _DP)
    bd1 = _pb(dec_n1_b, _DP)
    d2 = _pw(dec_n2_w, _DP, 32)
    bd2 = _pb(dec_n2_b, 32)

    n0, p1_0, p2_0 = _node_encode(
        nodes, enc_n1_w, enc_n1_b.reshape(1, 32),
        _pw(enc_n2_w, 32, _DP), _pb(enc_n2_b, _DP), g0w1, g0w2)

    G0 = _gather_pair(p1_0, p2_0, rw, sw)
    contrib1, T1 = _edge_layer0(
        G0, edge_features, edge_weights,
        enc_e1_w, enc_e1_b.reshape(1, 32),
        _pw(enc_e2_w, 32, _DP), _pb(enc_e2_b, _DP),
        g0w3, g0b3, g1w3, g1b3)
    parts0 = _scatter_add(contrib1, rw)
    n1, p1_1, p2_1 = _node_update(n0, parts0, g0n1, g0nb, g0n2, g1w1, g1w2)

    G1 = _gather_pair(p1_1, p2_1, rw, sw)
    contrib2, out_e = _edge_layer1(
        G1, T1, edge_weights, d1, bd1, d2, bd2, out_e_w, _pb(out_e_b, 3))
    parts1 = _scatter_add(contrib2, rw)

    out_n = _node_final(
        n1, parts1, g1n1, g1nb, g1n2, d1, bd1, d2, bd2,
        out_n_w, _pb(out_n_b, 3))

    return out_n, out_e


# speedup vs baseline: 1.3339x; 1.0020x over previous
import functools

import jax
import jax.numpy as jnp
from jax import lax
from jax.experimental import pallas as pl
from jax.experimental.pallas import tpu as pltpu
from jax.experimental.pallas import tpu_sc as plsc

_N = 10000
_E = 320000
_DP = 128
_NP = 10240
_W = 128
_NWIN = _E // _W
_WPW = 80
_IPAD = _WPW * 32
_BE = 4000


def _relu(x):
    return jnp.maximum(x, 0.0)


def _dot(a, b):
    return jnp.dot(a, b, preferred_element_type=jnp.float32)


def _pw(w, rows, cols):
    return jnp.pad(w, ((0, rows - w.shape[0]), (0, cols - w.shape[1])))


def _pb(b, n):
    return jnp.pad(b, (0, n - b.shape[0])).reshape(1, n)


def _sc_mesh():
    return plsc.VectorSubcoreMesh(core_axis_name="core", subcore_axis_name="subcore")


def _gather_pair(p1, p2, r2d, s2d):
    rows_t = pltpu.VMEM((_W, _DP), jnp.float32)

    @functools.partial(
        pl.kernel,
        out_type=jax.ShapeDtypeStruct((_E, _DP), jnp.float32),
        mesh=_sc_mesh(),
        scratch_types=[
            pltpu.VMEM((_WPW, _W), jnp.int32),
            pltpu.VMEM((_WPW, _W), jnp.int32),
            rows_t, rows_t, rows_t, rows_t,
            rows_t, rows_t,
            pltpu.SemaphoreType.DMA, pltpu.SemaphoreType.DMA,
            pltpu.SemaphoreType.DMA, pltpu.SemaphoreType.DMA,
            pltpu.SemaphoreType.DMA, pltpu.SemaphoreType.DMA,
            pltpu.SemaphoreType.DMA,
        ],
    )
    def k(p1_hbm, p2_hbm, r_hbm, s_hbm, g_hbm,
          ir, isb, r1a, r2a, r1b, r2b, oa, ob,
          s1a, s2a, s1b, s2b, soa, sob, sp):
        cid = lax.axis_index("core")
        sid = lax.axis_index("subcore")
        wid = sid * 2 + cid
        base = wid * _WPW

        pltpu.async_copy(r_hbm.at[pl.ds(base, _WPW)], ir, sp).wait()
        pltpu.async_copy(s_hbm.at[pl.ds(base, _WPW)], isb, sp).wait()

        bufs = ((r1a, r2a, s1a, s2a, oa, soa),
                (r1b, r2b, s1b, s2b, ob, sob))

        def issue(t, r1, r2, s1, s2):
            @pl.when(jnp.logical_and(t < _WPW, base + t < _NWIN))
            def _():
                pltpu.async_copy(p1_hbm.at[ir.at[t]], r1, s1)
                pltpu.async_copy(p2_hbm.at[isb.at[t]], r2, s2)

        def wait_rows(t, r1, r2, s1, s2):
            @pl.when(base + t < _NWIN)
            def _():
                pltpu.make_async_copy(p1_hbm.at[ir.at[t]], r1, s1).wait()
                pltpu.make_async_copy(p2_hbm.at[isb.at[t]], r2, s2).wait()

        def wait_out(t, o, so):
            @pl.when(jnp.logical_and(t >= 0, base + t < _NWIN))
            def _():
                pltpu.make_async_copy(
                    o, g_hbm.at[pl.ds((base + t) * _W, _W)], so).wait()

        issue(0, r1a, r2a, s1a, s2a)

        @pl.loop(0, _WPW // 2)
        def _(u):
            for half in range(2):
                t = u * 2 + half
                r1, r2, s1, s2, o, so = bufs[half]
                n1, n2, ns1, ns2, _, _ = bufs[1 - half]
                issue(t + 1, n1, n2, ns1, ns2)
                wait_rows(t, r1, r2, s1, s2)
                wait_out(t - 2, o, so)

                @pl.when(base + t < _NWIN)
                def _():
                    @pl.loop(0, _W, step=4)
                    def _(i):
                        for dr in range(4):
                            for c in range(_DP // 16):
                                slc = (pl.ds(i + dr, 1), pl.ds(c * 16, 16))
                                o.at[slc][...] = (r1.at[slc][...]
                                                  + r2.at[slc][...])

                    pltpu.async_copy(
                        o, g_hbm.at[pl.ds((base + t) * _W, _W)], so)

        wait_out(_WPW - 2, oa, soa)
        wait_out(_WPW - 1, ob, sob)

    return k(p1, p2, r2d, s2d)


def _scatter_add(contrib, r1d):
    @functools.partial(
        pl.kernel,
        out_type=jax.ShapeDtypeStruct((2, _NP, _DP), jnp.float32),
        mesh=_sc_mesh(),
        scratch_types=[
            pltpu.VMEM((32, _DP), jnp.float32),
            pltpu.VMEM((_W, _DP), jnp.float32),
            pltpu.VMEM((_W, _DP), jnp.float32),
            pltpu.VMEM((_WPW, _W), jnp.int32),
            pltpu.VMEM_SHARED((_NP, _DP), jnp.float32),
            pltpu.SemaphoreType.DMA, pltpu.SemaphoreType.DMA,
            pltpu.SemaphoreType.DMA,
        ],
    )
    def k(c_hbm, r_hbm, out_hbm, zbuf, cba, cbb, ir, agg, sa, sb, sp):
        cid = lax.axis_index("core")
        sid = lax.axis_index("subcore")
        wid = sid * 2 + cid
        base = wid * _WPW

        pltpu.async_copy(r_hbm.at[pl.ds(base, _WPW)], ir, sp).wait()

        @pl.loop(0, 32)
        def _(i):
            for c in range(_DP // 16):
                zbuf.at[pl.ds(i, 1), pl.ds(c * 16, 16)][...] = jnp.zeros(
                    (1, 16), jnp.float32)

        @pl.loop(0, 20)
        def _(j):
            pltpu.async_copy(zbuf, agg.at[pl.ds(sid * 640 + j * 32, 32)], sp)

        @pl.loop(0, 20)
        def _(j):
            pltpu.make_async_copy(
                zbuf, agg.at[pl.ds(sid * 640 + j * 32, 32)], sp).wait()

        plsc.subcore_barrier()

        def issue(t, cb, sem):
            @pl.when(jnp.logical_and(t < _WPW, base + t < _NWIN))
            def _():
                pltpu.async_copy(
                    c_hbm.at[pl.ds((base + t) * _W, _W)], cb, sem)

        def wait_load(t, cb, sem):
            @pl.when(base + t < _NWIN)
            def _():
                pltpu.make_async_copy(
                    c_hbm.at[pl.ds((base + t) * _W, _W)], cb, sem).wait()

        issue(0, cba, sa)

        @pl.loop(0, _WPW // 2)
        def _(u):
            for half in range(2):
                t = u * 2 + half
                cb, sem = (cba, sa) if half == 0 else (cbb, sb)
                ncb, nsem = (cbb, sb) if half == 0 else (cba, sa)
                issue(t + 1, ncb, nsem)
                wait_load(t, cb, sem)

                @pl.when(base + t < _NWIN)
                def _():
                    pltpu.sync_copy(cb, agg.at[ir.at[t]], add=True)

        plsc.subcore_barrier()

        pltpu.sync_copy(agg.at[pl.ds(sid * 640, 640)],
                        out_hbm.at[cid, pl.ds(sid * 640, 640)])

    return k(contrib, r1d)


def _full(shape):
    nd = len(shape)
    return pl.BlockSpec(shape, lambda i: (0,) * nd)


def _node_encode(nodes, w1, b1, w2, b2, wp1, wp2):
    def body(x, w1r, b1r, w2r, b2r, wp1r, wp2r, n0o, p1o, p2o):
        h = _relu(_dot(x[...], w1r[...]) + b1r[...])
        n0 = _relu(_dot(h, w2r[...]) + b2r[...])
        n0o[...] = n0
        p1o[...] = _dot(n0, wp1r[...])
        p2o[...] = _dot(n0, wp2r[...])

    out = jax.ShapeDtypeStruct((_N, _DP), jnp.float32)
    return pl.pallas_call(
        body,
        out_shape=[out, out, out],
    )(nodes, w1, b1, w2, b2, wp1, wp2)


def _node_update(n_prev, parts, wn1, bn, wn2, wp1, wp2):
    def body(n, pr, wn1r, bnr, wn2r, wp1r, wp2r, n1o, p1o, p2o):
        agg = pr[0, : _N, :] + pr[1, : _N, :]
        n1 = _relu(_dot(n[...], wn1r[...]) + _dot(agg, wn2r[...]) + bnr[...])
        n1o[...] = n1
        p1o[...] = _dot(n1, wp1r[...])
        p2o[...] = _dot(n1, wp2r[...])

    out = jax.ShapeDtypeStruct((_N, _DP), jnp.float32)
    return pl.pallas_call(
        body,
        out_shape=[out, out, out],
    )(n_prev, parts, wn1, bn, wn2, wp1, wp2)


def _node_final(n_prev, parts, wn1, bn, wn2, d1, bd1, d2, bd2, won, bon):
    def body(n, pr, wn1r, bnr, wn2r, d1r, bd1r, d2r, bd2r, wonr, bonr, outo):
        agg = pr[0, : _N, :] + pr[1, : _N, :]
        n2 = _relu(_dot(n[...], wn1r[...]) + _dot(agg, wn2r[...]) + bnr[...])
        dn = _relu(_dot(n2, d1r[...]) + bd1r[...])
        dn = _relu(_dot(dn, d2r[...]) + bd2r[...])
        outo[...] = _dot(dn, wonr[...]) + bonr[...]

    return pl.pallas_call(
        body,
        out_shape=jax.ShapeDtypeStruct((_N, 3), jnp.float32),
    )(n_prev, parts, wn1, bn, wn2, d1, bd1, d2, bd2, won, bon)


def _edge_layer0(G0, efeat, ew, we1, be1, we2, be2, w30, b30, w31, b31):
    bf = jnp.bfloat16

    def body(g, f, w, we1r, be1r, we2r, be2r, w30r, b30r, w31r, b31r, co, t1o):
        h = _relu(_dot(f[...].astype(bf), we1r[...].astype(bf)) + be1r[...])
        ef0 = _relu(_dot(h.astype(bf), we2r[...].astype(bf)) + be2r[...])
        ef1 = _relu(g[...] + _dot(ef0.astype(bf), w30r[...].astype(bf))
                    + b30r[...])
        co[...] = ef1 * w[...]
        t1o[...] = (_dot(ef1.astype(bf), w31r[...].astype(bf))
                    + b31r[...]).astype(bf)

    return pl.pallas_call(
        body,
        grid=(_E // _BE,),
        in_specs=[
            pl.BlockSpec((_BE, _DP), lambda i: (i, 0)),
            pl.BlockSpec((_BE, 16), lambda i: (i, 0)),
            pl.BlockSpec((_BE, 1), lambda i: (i, 0)),
            _full((16, 32)), _full((1, 32)),
            _full((32, _DP)), _full((1, _DP)),
            _full((_DP, _DP)), _full((1, _DP)),
            _full((_DP, _DP)), _full((1, _DP)),
        ],
        out_specs=[
            pl.BlockSpec((_BE, _DP), lambda i: (i, 0)),
            pl.BlockSpec((_BE, _DP), lambda i: (i, 0)),
        ],
        out_shape=[
            jax.ShapeDtypeStruct((_E, _DP), jnp.float32),
            jax.ShapeDtypeStruct((_E, _DP), jnp.bfloat16),
        ],
    )(G0, efeat, ew, we1, be1, we2, be2, w30, b30, w31, b31)


def _edge_layer1(G1, T1, ew, d1, bd1, d2, bd2, woe, boe):
    bf = jnp.bfloat16

    def body(g, t, w, d1r, bd1r, d2r, bd2r, woer, boer, co, oeo):
        ef2 = _relu(g[...] + t[...].astype(jnp.float32))
        co[...] = ef2 * w[...]
        de = _relu(_dot(ef2.astype(bf), d1r[...].astype(bf)) + bd1r[...])
        de = _relu(_dot(de.astype(bf), d2r[...].astype(bf)) + bd2r[...])
        oeo[...] = _dot(de.astype(bf), woer[...].astype(bf)) + boer[...]

    return pl.pallas_call(
        body,
        grid=(_E // _BE,),
        in_specs=[
            pl.BlockSpec((_BE, _DP), lambda i: (i, 0)),
            pl.BlockSpec((_BE, _DP), lambda i: (i, 0)),
            pl.BlockSpec((_BE, 1), lambda i: (i, 0)),
            _full((_DP, _DP)), _full((1, _DP)),
            _full((_DP, 32)), _full((1, 32)),
            _full((32, 3)), _full((1, 3)),
        ],
        out_specs=[
            pl.BlockSpec((_BE, _DP), lambda i: (i, 0)),
            pl.BlockSpec((_BE, 3), lambda i: (i, 0)),
        ],
        out_shape=[
            jax.ShapeDtypeStruct((_E, _DP), jnp.float32),
            jax.ShapeDtypeStruct((_E, 3), jnp.float32),
        ],
    )(G1, T1, ew, d1, bd1, d2, bd2, woe, boe)


def kernel(nodes, edge_features, edges, edge_weights,
           enc_n1_w, enc_n1_b, enc_n2_w, enc_n2_b,
           enc_e1_w, enc_e1_b, enc_e2_w, enc_e2_b,
           g0_edge_w, g0_edge_b, g0_node_w, g0_node_b,
           g1_edge_w, g1_edge_b, g1_node_w, g1_node_b,
           dec_n1_w, dec_n1_b, dec_n2_w, dec_n2_b,
           out_n_w, out_n_b, out_e_w, out_e_b):
    r = edges[:, 1]
    s = edges[:, 0]
    pad = _IPAD * _W - _E
    rw = jnp.pad(r, (0, pad)).reshape(_IPAD, _W)
    sw = jnp.pad(s, (0, pad)).reshape(_IPAD, _W)

    g0w1 = _pw(g0_edge_w[0:72], _DP, _DP)
    g0w2 = _pw(g0_edge_w[72:144], _DP, _DP)
    g0w3 = _pw(g0_edge_w[144:216], _DP, _DP)
    g0b3 = _pb(g0_edge_b, _DP)
    g1w1 = _pw(g1_edge_w[0:72], _DP, _DP)
    g1w2 = _pw(g1_edge_w[72:144], _DP, _DP)
    g1w3 = _pw(g1_edge_w[144:216], _DP, _DP)
    g1b3 = _pb(g1_edge_b, _DP)
    g0n1 = _pw(g0_node_w[0:72], _DP, _DP)
    g0n2 = _pw(g0_node_w[72:144], _DP, _DP)
    g0nb = _pb(g0_node_b, _DP)
    g1n1 = _pw(g1_node_w[0:72], _DP, _DP)
    g1n2 = _pw(g1_node_w[72:144], _DP, _DP)
    g1nb = _pb(g1_node_b, _DP)
    d1 = _pw(dec_n1_w, _DP, _DP)
    bd1 = _pb(dec_n1_b, _DP)
    d2 = _pw(dec_n2_w, _DP, 32)
    bd2 = _pb(dec_n2_b, 32)

    n0, p1_0, p2_0 = _node_encode(
        nodes, enc_n1_w, enc_n1_b.reshape(1, 32),
        _pw(enc_n2_w, 32, _DP), _pb(enc_n2_b, _DP), g0w1, g0w2)

    G0 = _gather_pair(p1_0, p2_0, rw, sw)
    contrib1, T1 = _edge_layer0(
        G0, edge_features, edge_weights,
        enc_e1_w, enc_e1_b.reshape(1, 32),
        _pw(enc_e2_w, 32, _DP), _pb(enc_e2_b, _DP),
        g0w3, g0b3, g1w3, g1b3)
    parts0 = _scatter_add(contrib1, rw)
    n1, p1_1, p2_1 = _node_update(n0, parts0, g0n1, g0nb, g0n2, g1w1, g1w2)

    G1 = _gather_pair(p1_1, p2_1, rw, sw)
    contrib2, out_e = _edge_layer1(
        G1, T1, edge_weights, d1, bd1, d2, bd2, out_e_w, _pb(out_e_b, 3))
    parts1 = _scatter_add(contrib2, rw)

    out_n = _node_final(
        n1, parts1, g1n1, g1nb, g1n2, d1, bd1, d2, bd2,
        out_n_w, _pb(out_n_b, 3))

    return out_n, out_e


# speedup vs baseline: 1.3480x; 1.0105x over previous
import functools

import jax
import jax.numpy as jnp
from jax import lax
from jax.experimental import pallas as pl
from jax.experimental.pallas import tpu as pltpu
from jax.experimental.pallas import tpu_sc as plsc

_N = 10000
_E = 320000
_DP = 128
_NP = 10240
_W = 128
_NWIN = _E // _W
_WPW = 80
_IPAD = _WPW * 32
_BE = 8000


def _relu(x):
    return jnp.maximum(x, 0.0)


def _dot(a, b):
    return jnp.dot(a, b, preferred_element_type=jnp.float32)


def _pw(w, rows, cols):
    return jnp.pad(w, ((0, rows - w.shape[0]), (0, cols - w.shape[1])))


def _pb(b, n):
    return jnp.pad(b, (0, n - b.shape[0])).reshape(1, n)


def _sc_mesh():
    return plsc.VectorSubcoreMesh(core_axis_name="core", subcore_axis_name="subcore")


def _sc_warmup(x):
    @functools.partial(
        pl.kernel,
        out_type=jax.ShapeDtypeStruct((32, 8), jnp.float32),
        mesh=_sc_mesh(),
        scratch_types=[pltpu.VMEM((8,), jnp.float32),
                       pltpu.SemaphoreType.DMA],
    )
    def k(x_hbm, o_hbm, buf, sem):
        cid = lax.axis_index("core")
        sid = lax.axis_index("subcore")
        wid = sid * 2 + cid
        pltpu.async_copy(x_hbm, buf, sem).wait()
        pltpu.async_copy(buf, o_hbm.at[wid], sem).wait()

    return k(x)


def _gather_pair(p1, p2, r2d, s2d):
    rows_t = pltpu.VMEM((_W, _DP), jnp.float32)

    @functools.partial(
        pl.kernel,
        out_type=jax.ShapeDtypeStruct((_E, _DP), jnp.float32),
        mesh=_sc_mesh(),
        scratch_types=[
            pltpu.VMEM((_WPW, _W), jnp.int32),
            pltpu.VMEM((_WPW, _W), jnp.int32),
            rows_t, rows_t, rows_t, rows_t,
            rows_t, rows_t,
            pltpu.SemaphoreType.DMA, pltpu.SemaphoreType.DMA,
            pltpu.SemaphoreType.DMA, pltpu.SemaphoreType.DMA,
            pltpu.SemaphoreType.DMA, pltpu.SemaphoreType.DMA,
            pltpu.SemaphoreType.DMA,
        ],
    )
    def k(p1_hbm, p2_hbm, r_hbm, s_hbm, g_hbm,
          ir, isb, r1a, r2a, r1b, r2b, oa, ob,
          s1a, s2a, s1b, s2b, soa, sob, sp):
        cid = lax.axis_index("core")
        sid = lax.axis_index("subcore")
        wid = sid * 2 + cid
        base = wid * _WPW

        pltpu.async_copy(r_hbm.at[pl.ds(base, _WPW)], ir, sp).wait()
        pltpu.async_copy(s_hbm.at[pl.ds(base, _WPW)], isb, sp).wait()

        bufs = ((r1a, r2a, s1a, s2a, oa, soa),
                (r1b, r2b, s1b, s2b, ob, sob))

        def issue(t, r1, r2, s1, s2):
            @pl.when(jnp.logical_and(t < _WPW, base + t < _NWIN))
            def _():
                pltpu.async_copy(p1_hbm.at[ir.at[t]], r1, s1)
                pltpu.async_copy(p2_hbm.at[isb.at[t]], r2, s2)

        def wait_rows(t, r1, r2, s1, s2):
            @pl.when(base + t < _NWIN)
            def _():
                pltpu.make_async_copy(p1_hbm.at[ir.at[t]], r1, s1).wait()
                pltpu.make_async_copy(p2_hbm.at[isb.at[t]], r2, s2).wait()

        def wait_out(t, o, so):
            @pl.when(jnp.logical_and(t >= 0, base + t < _NWIN))
            def _():
                pltpu.make_async_copy(
                    o, g_hbm.at[pl.ds((base + t) * _W, _W)], so).wait()

        issue(0, r1a, r2a, s1a, s2a)

        @pl.loop(0, _WPW // 2)
        def _(u):
            for half in range(2):
                t = u * 2 + half
                r1, r2, s1, s2, o, so = bufs[half]
                n1, n2, ns1, ns2, _, _ = bufs[1 - half]
                issue(t + 1, n1, n2, ns1, ns2)
                wait_rows(t, r1, r2, s1, s2)
                wait_out(t - 2, o, so)

                @pl.when(base + t < _NWIN)
                def _():
                    @pl.loop(0, _W, step=4)
                    def _(i):
                        for dr in range(4):
                            for c in range(_DP // 16):
                                slc = (pl.ds(i + dr, 1), pl.ds(c * 16, 16))
                                o.at[slc][...] = (r1.at[slc][...]
                                                  + r2.at[slc][...])

                    pltpu.async_copy(
                        o, g_hbm.at[pl.ds((base + t) * _W, _W)], so)

        wait_out(_WPW - 2, oa, soa)
        wait_out(_WPW - 1, ob, sob)

    return k(p1, p2, r2d, s2d)


def _scatter_add(contrib, r1d):
    @functools.partial(
        pl.kernel,
        out_type=jax.ShapeDtypeStruct((2, _NP, _DP), jnp.float32),
        mesh=_sc_mesh(),
        scratch_types=[
            pltpu.VMEM((32, _DP), jnp.float32),
            pltpu.VMEM((_W, _DP), jnp.float32),
            pltpu.VMEM((_W, _DP), jnp.float32),
            pltpu.VMEM((_WPW, _W), jnp.int32),
            pltpu.VMEM_SHARED((_NP, _DP), jnp.float32),
            pltpu.SemaphoreType.DMA, pltpu.SemaphoreType.DMA,
            pltpu.SemaphoreType.DMA,
        ],
    )
    def k(c_hbm, r_hbm, out_hbm, zbuf, cba, cbb, ir, agg, sa, sb, sp):
        cid = lax.axis_index("core")
        sid = lax.axis_index("subcore")
        wid = sid * 2 + cid
        base = wid * _WPW

        pltpu.async_copy(r_hbm.at[pl.ds(base, _WPW)], ir, sp).wait()

        @pl.loop(0, 32)
        def _(i):
            for c in range(_DP // 16):
                zbuf.at[pl.ds(i, 1), pl.ds(c * 16, 16)][...] = jnp.zeros(
                    (1, 16), jnp.float32)

        @pl.loop(0, 20)
        def _(j):
            pltpu.async_copy(zbuf, agg.at[pl.ds(sid * 640 + j * 32, 32)], sp)

        @pl.loop(0, 20)
        def _(j):
            pltpu.make_async_copy(
                zbuf, agg.at[pl.ds(sid * 640 + j * 32, 32)], sp).wait()

        plsc.subcore_barrier()

        def issue(t, cb, sem):
            @pl.when(jnp.logical_and(t < _WPW, base + t < _NWIN))
            def _():
                pltpu.async_copy(
                    c_hbm.at[pl.ds((base + t) * _W, _W)], cb, sem)

        def wait_load(t, cb, sem):
            @pl.when(base + t < _NWIN)
            def _():
                pltpu.make_async_copy(
                    c_hbm.at[pl.ds((base + t) * _W, _W)], cb, sem).wait()

        issue(0, cba, sa)

        @pl.loop(0, _WPW // 2)
        def _(u):
            for half in range(2):
                t = u * 2 + half
                cb, sem = (cba, sa) if half == 0 else (cbb, sb)
                ncb, nsem = (cbb, sb) if half == 0 else (cba, sa)
                issue(t + 1, ncb, nsem)
                wait_load(t, cb, sem)

                @pl.when(base + t < _NWIN)
                def _():
                    pltpu.sync_copy(cb, agg.at[ir.at[t]], add=True)

        plsc.subcore_barrier()

        pltpu.sync_copy(agg.at[pl.ds(sid * 640, 640)],
                        out_hbm.at[cid, pl.ds(sid * 640, 640)])

    return k(contrib, r1d)


def _full(shape):
    nd = len(shape)
    return pl.BlockSpec(shape, lambda i: (0,) * nd)


def _node_encode(nodes, w1, b1, w2, b2, wp1, wp2):
    def body(x, w1r, b1r, w2r, b2r, wp1r, wp2r, n0o, p1o, p2o):
        h = _relu(_dot(x[...], w1r[...]) + b1r[...])
        n0 = _relu(_dot(h, w2r[...]) + b2r[...])
        n0o[...] = n0
        p1o[...] = _dot(n0, wp1r[...])
        p2o[...] = _dot(n0, wp2r[...])

    out = jax.ShapeDtypeStruct((_N, _DP), jnp.float32)
    return pl.pallas_call(
        body,
        out_shape=[out, out, out],
    )(nodes, w1, b1, w2, b2, wp1, wp2)


def _node_update(n_prev, parts, wn1, bn, wn2, wp1, wp2):
    def body(n, pr, wn1r, bnr, wn2r, wp1r, wp2r, n1o, p1o, p2o):
        agg = pr[0, : _N, :] + pr[1, : _N, :]
        n1 = _relu(_dot(n[...], wn1r[...]) + _dot(agg, wn2r[...]) + bnr[...])
        n1o[...] = n1
        p1o[...] = _dot(n1, wp1r[...])
        p2o[...] = _dot(n1, wp2r[...])

    out = jax.ShapeDtypeStruct((_N, _DP), jnp.float32)
    return pl.pallas_call(
        body,
        out_shape=[out, out, out],
    )(n_prev, parts, wn1, bn, wn2, wp1, wp2)


def _node_final(n_prev, parts, wn1, bn, wn2, d1, bd1, d2, bd2, won, bon):
    def body(n, pr, wn1r, bnr, wn2r, d1r, bd1r, d2r, bd2r, wonr, bonr, outo):
        agg = pr[0, : _N, :] + pr[1, : _N, :]
        n2 = _relu(_dot(n[...], wn1r[...]) + _dot(agg, wn2r[...]) + bnr[...])
        dn = _relu(_dot(n2, d1r[...]) + bd1r[...])
        dn = _relu(_dot(dn, d2r[...]) + bd2r[...])
        outo[...] = _dot(dn, wonr[...]) + bonr[...]

    return pl.pallas_call(
        body,
        out_shape=jax.ShapeDtypeStruct((_N, 3), jnp.float32),
    )(n_prev, parts, wn1, bn, wn2, d1, bd1, d2, bd2, won, bon)


def _edge_layer0(G0, efeat, ew, we1, be1, we2, be2, w30, b30, w31, b31):
    bf = jnp.bfloat16

    def body(g, f, w, we1r, be1r, we2r, be2r, w30r, b30r, w31r, b31r, co, t1o):
        h = _relu(_dot(f[...].astype(bf), we1r[...].astype(bf)) + be1r[...])
        ef0 = _relu(_dot(h.astype(bf), we2r[...].astype(bf)) + be2r[...])
        ef1 = _relu(g[...] + _dot(ef0.astype(bf), w30r[...].astype(bf))
                    + b30r[...])
        co[...] = ef1 * w[...]
        t1o[...] = (_dot(ef1.astype(bf), w31r[...].astype(bf))
                    + b31r[...]).astype(bf)

    return pl.pallas_call(
        body,
        grid=(_E // _BE,),
        in_specs=[
            pl.BlockSpec((_BE, _DP), lambda i: (i, 0)),
            pl.BlockSpec((_BE, 16), lambda i: (i, 0)),
            pl.BlockSpec((_BE, 1), lambda i: (i, 0)),
            _full((16, 32)), _full((1, 32)),
            _full((32, _DP)), _full((1, _DP)),
            _full((_DP, _DP)), _full((1, _DP)),
            _full((_DP, _DP)), _full((1, _DP)),
        ],
        out_specs=[
            pl.BlockSpec((_BE, _DP), lambda i: (i, 0)),
            pl.BlockSpec((_BE, _DP), lambda i: (i, 0)),
        ],
        out_shape=[
            jax.ShapeDtypeStruct((_E, _DP), jnp.float32),
            jax.ShapeDtypeStruct((_E, _DP), jnp.bfloat16),
        ],
    )(G0, efeat, ew, we1, be1, we2, be2, w30, b30, w31, b31)


def _edge_layer1(G1, T1, ew, d1, bd1, d2, bd2, woe, boe):
    bf = jnp.bfloat16

    def body(g, t, w, d1r, bd1r, d2r, bd2r, woer, boer, co, oeo):
        ef2 = _relu(g[...] + t[...].astype(jnp.float32))
        co[...] = ef2 * w[...]
        de = _relu(_dot(ef2.astype(bf), d1r[...].astype(bf)) + bd1r[...])
        de = _relu(_dot(de.astype(bf), d2r[...].astype(bf)) + bd2r[...])
        oeo[...] = _dot(de.astype(bf), woer[...].astype(bf)) + boer[...]

    return pl.pallas_call(
        body,
        grid=(_E // _BE,),
        in_specs=[
            pl.BlockSpec((_BE, _DP), lambda i: (i, 0)),
            pl.BlockSpec((_BE, _DP), lambda i: (i, 0)),
            pl.BlockSpec((_BE, 1), lambda i: (i, 0)),
            _full((_DP, _DP)), _full((1, _DP)),
            _full((_DP, 32)), _full((1, 32)),
            _full((32, 3)), _full((1, 3)),
        ],
        out_specs=[
            pl.BlockSpec((_BE, _DP), lambda i: (i, 0)),
            pl.BlockSpec((_BE, 3), lambda i: (i, 0)),
        ],
        out_shape=[
            jax.ShapeDtypeStruct((_E, _DP), jnp.float32),
            jax.ShapeDtypeStruct((_E, 3), jnp.float32),
        ],
    )(G1, T1, ew, d1, bd1, d2, bd2, woe, boe)


def kernel(nodes, edge_features, edges, edge_weights,
           enc_n1_w, enc_n1_b, enc_n2_w, enc_n2_b,
           enc_e1_w, enc_e1_b, enc_e2_w, enc_e2_b,
           g0_edge_w, g0_edge_b, g0_node_w, g0_node_b,
           g1_edge_w, g1_edge_b, g1_node_w, g1_node_b,
           dec_n1_w, dec_n1_b, dec_n2_w, dec_n2_b,
           out_n_w, out_n_b, out_e_w, out_e_b):
    r = edges[:, 1]
    s = edges[:, 0]
    pad = _IPAD * _W - _E
    rw = jnp.pad(r, (0, pad)).reshape(_IPAD, _W)
    sw = jnp.pad(s, (0, pad)).reshape(_IPAD, _W)

    wout = _sc_warmup(edge_weights[:8, 0])
    rw = lax.optimization_barrier((rw, wout))[0]

    g0w1 = _pw(g0_edge_w[0:72], _DP, _DP)
    g0w2 = _pw(g0_edge_w[72:144], _DP, _DP)
    g0w3 = _pw(g0_edge_w[144:216], _DP, _DP)
    g0b3 = _pb(g0_edge_b, _DP)
    g1w1 = _pw(g1_edge_w[0:72], _DP, _DP)
    g1w2 = _pw(g1_edge_w[72:144], _DP, _DP)
    g1w3 = _pw(g1_edge_w[144:216], _DP, _DP)
    g1b3 = _pb(g1_edge_b, _DP)
    g0n1 = _pw(g0_node_w[0:72], _DP, _DP)
    g0n2 = _pw(g0_node_w[72:144], _DP, _DP)
    g0nb = _pb(g0_node_b, _DP)
    g1n1 = _pw(g1_node_w[0:72], _DP, _DP)
    g1n2 = _pw(g1_node_w[72:144], _DP, _DP)
    g1nb = _pb(g1_node_b, _DP)
    d1 = _pw(dec_n1_w, _DP, _DP)
    bd1 = _pb(dec_n1_b, _DP)
    d2 = _pw(dec_n2_w, _DP, 32)
    bd2 = _pb(dec_n2_b, 32)

    n0, p1_0, p2_0 = _node_encode(
        nodes, enc_n1_w, enc_n1_b.reshape(1, 32),
        _pw(enc_n2_w, 32, _DP), _pb(enc_n2_b, _DP), g0w1, g0w2)

    G0 = _gather_pair(p1_0, p2_0, rw, sw)
    contrib1, T1 = _edge_layer0(
        G0, edge_features, edge_weights,
        enc_e1_w, enc_e1_b.reshape(1, 32),
        _pw(enc_e2_w, 32, _DP), _pb(enc_e2_b, _DP),
        g0w3, g0b3, g1w3, g1b3)
    parts0 = _scatter_add(contrib1, rw)
    n1, p1_1, p2_1 = _node_update(n0, parts0, g0n1, g0nb, g0n2, g1w1, g1w2)

    G1 = _gather_pair(p1_1, p2_1, rw, sw)
    contrib2, out_e = _edge_layer1(
        G1, T1, edge_weights, d1, bd1, d2, bd2, out_e_w, _pb(out_e_b, 3))
    parts1 = _scatter_add(contrib2, rw)

    out_n = _node_final(
        n1, parts1, g1n1, g1nb, g1n2, d1, bd1, d2, bd2,
        out_n_w, _pb(out_n_b, 3))

    return out_n, out_e
